# baseline (speedup 1.0000x reference)
import functools

import jax
import jax.numpy as jnp
from jax import lax
from jax.experimental import pallas as pl
from jax.experimental.pallas import tpu as pltpu
from jax.experimental.pallas import tpu_sc as plsc

_B = 16384
_D = 16
_W = 7
_ROW_SHIFT = 1
_NW = 32
_BPW = _B // _NW
_CH = _BPW // 16
_CROWS = 16 * _W
_ROWS = _BPW * _W


def _build_gather():
    mesh = plsc.VectorSubcoreMesh(core_axis_name="c", subcore_axis_name="s")

    @functools.partial(
        pl.kernel,
        mesh=mesh,
        compiler_params=pltpu.CompilerParams(
            use_tc_tiling_on_sc=False, needs_layout_passes=False
        ),
        out_type=jax.ShapeDtypeStruct((_B * _W, _D), jnp.float32),
        scratch_types=[
            pltpu.VMEM((_BPW,), jnp.int32),
            pltpu.VMEM((_ROWS,), jnp.int32),
            pltpu.VMEM((_ROWS, _D), jnp.float32),
            pltpu.SemaphoreType.DMA,
        ],
    )
    def gather_kernel(idx_hbm, emb_hbm, out_hbm, idx_v, exp_v, rows_v, sem):
        wid = lax.axis_index("s") * 2 + lax.axis_index("c")
        base = wid * _BPW
        pltpu.sync_copy(idx_hbm.at[pl.ds(base, _BPW)], idx_v)

        col0 = lax.iota(jnp.int32, 16) * _W

        def expand_and_fire(c, carry):
            x = idx_v[pl.ds(c * 16, 16)]
            pos0 = col0 + c * _CROWS
            for j in range(_W):
                plsc.store_scatter(exp_v, [pos0 + j], x + (_ROW_SHIFT + j))
            pltpu.async_copy(
                emb_hbm.at[exp_v.at[pl.ds(c * _CROWS, _CROWS)]],
                rows_v.at[pl.ds(c * _CROWS, _CROWS)],
                sem,
            )
            return carry

        lax.fori_loop(0, _CH, expand_and_fire, 0)
        pltpu.make_async_copy(emb_hbm.at[exp_v], rows_v, sem).wait()
        pltpu.sync_copy(rows_v, out_hbm.at[pl.ds(base * _W, _ROWS)])

    return gather_kernel


_TBLK = 512
_TROWS = 1000009
_TGRID = -(-_TROWS // _TBLK)
_TPAD = _TGRID * _TBLK


def _tc_transpose_kernel(x_ref, o_ref):
    o_ref[...] = x_ref[...].T


def _tc_transpose():
    return pl.pallas_call(
        _tc_transpose_kernel,
        grid=(_TGRID,),
        in_specs=[pl.BlockSpec((_D, _TBLK), lambda i: (0, i))],
        out_specs=pl.BlockSpec((_TBLK, _D), lambda i: (i, 0)),
        out_shape=jax.ShapeDtypeStruct((_TPAD, _D), jnp.float32),
    )


def kernel(inputs, embedding):
    table_rm = _tc_transpose()(embedding.T)
    out2d = _build_gather()(inputs.astype(jnp.int32), table_rm)
    return out2d.reshape(_B, _W, _D)


# speedup vs baseline: 1.0871x; 1.0871x over previous
import functools

import jax
import jax.numpy as jnp
from jax import lax
from jax.experimental import pallas as pl
from jax.experimental.pallas import tpu as pltpu
from jax.experimental.pallas import tpu_sc as plsc

_B = 16384
_D = 16
_W = 7
_ROW_SHIFT = 1
_NW = 32
_BPW = _B // _NW
_NROWS = 1000009
_TROWS = (_NROWS * _D) // 16
_HB = 128
_NBATCH = _BPW // _HB
_RPB = _HB * 2 * _D
_GC = 128
_NG = _RPB // _GC
_WORDS = _BPW * _W * _D


def _build_gather():
    mesh = plsc.VectorSubcoreMesh(core_axis_name="c", subcore_axis_name="s")

    @functools.partial(
        pl.kernel,
        mesh=mesh,
        compiler_params=pltpu.CompilerParams(
            use_tc_tiling_on_sc=False, needs_layout_passes=False
        ),
        out_type=jax.ShapeDtypeStruct((_B * _W * _D,), jnp.float32),
        scratch_types=[
            pltpu.VMEM((_BPW + 16,), jnp.int32),
            pltpu.VMEM((_RPB,), jnp.int32),
            pltpu.VMEM((_RPB, _D), jnp.float32),
            pltpu.VMEM((_WORDS,), jnp.float32),
            pltpu.SemaphoreType.DMA,
        ],
    )
    def gather_kernel(idx_hbm, tab_hbm, out_hbm, idx_v, exp_v, buf_v, rows_v,
                      sem):
        wid = lax.axis_index("s") * 2 + lax.axis_index("c")
        base = wid * _BPW
        pltpu.sync_copy(idx_hbm.at[pl.ds(base, _BPW)], idx_v.at[pl.ds(0, _BPW)])

        lanes = lax.iota(jnp.int32, 16)
        ubase = lanes * 62500
        l9 = lanes * 9
        pos_pair = lanes * 2

        for h in range(_NBATCH):

            def expand(w, carry, h=h):
                x = idx_v[pl.ds(h * _HB + w, 16)]
                r1 = x[0] + _ROW_SHIFT
                u = ubase + ((l9 + r1) >> 4)
                p0 = w * (2 * _D) + pos_pair
                plsc.store_scatter(exp_v, [p0], u)
                plsc.store_scatter(exp_v, [p0 + 1], u + 1)
                return carry

            lax.fori_loop(0, _HB, expand, 0)

            def fire(g, carry):
                pltpu.async_copy(
                    tab_hbm.at[exp_v.at[pl.ds(g * _GC, _GC)]],
                    buf_v.at[pl.ds(g * _GC, _GC)],
                    sem,
                )
                return carry

            lax.fori_loop(0, _NG, fire, 0)
            pltpu.make_async_copy(
                tab_hbm.at[pl.ds(0, _RPB)], buf_v, sem
            ).wait()

            def assemble(w, carry, h=h):
                x = idx_v[pl.ds(h * _HB + w, 16)]
                r1 = x[0] + _ROW_SHIFT
                off0 = l9 + r1
                row0 = w * (2 * _D) + pos_pair
                obase = (h * _HB + w) * _W * _D
                for j in range(_W):
                    off = off0 + j
                    rows_v[pl.ds(obase + j * _D, _D)] = plsc.load_gather(
                        buf_v, [row0 + ((off >> 4) - (off0 >> 4)), off & 15]
                    )
                return carry

            lax.fori_loop(0, _HB, assemble, 0)

        pltpu.sync_copy(rows_v, out_hbm.at[pl.ds(base * _W * _D, _WORDS)])

    return gather_kernel


def kernel(inputs, embedding):
    table = embedding.T.reshape(_TROWS, _D)
    flat = _build_gather()(inputs.astype(jnp.int32), table)
    return flat.reshape(_B, _W, _D)


# speedup vs baseline: 2.9414x; 2.7057x over previous
import functools

import jax
import jax.numpy as jnp
from jax import lax
from jax.experimental import pallas as pl
from jax.experimental.pallas import tpu as pltpu
from jax.experimental.pallas import tpu_sc as plsc

_B = 16384
_D = 16
_W = 7
_ROW_SHIFT = 1
_NW = 32
_BPW = _B // _NW
_CH = _BPW // 16
_CROWS = 16 * _W
_ROWS = _BPW * _W


def _build_gather():
    mesh = plsc.VectorSubcoreMesh(core_axis_name="c", subcore_axis_name="s")

    @functools.partial(
        pl.kernel,
        mesh=mesh,
        compiler_params=pltpu.CompilerParams(
            use_tc_tiling_on_sc=False, needs_layout_passes=False
        ),
        out_type=jax.ShapeDtypeStruct((_B * _W, _D), jnp.float32),
        scratch_types=[
            pltpu.VMEM((_BPW,), jnp.int32),
            pltpu.VMEM((_ROWS,), jnp.int32),
            pltpu.VMEM((_ROWS, _D), jnp.float32),
            pltpu.SemaphoreType.DMA,
        ],
    )
    def gather_kernel(idx_hbm, emb_hbm, out_hbm, idx_v, exp_v, rows_v, sem):
        wid = lax.axis_index("s") * 2 + lax.axis_index("c")
        base = wid * _BPW
        pltpu.sync_copy(idx_hbm.at[pl.ds(base, _BPW)], idx_v)

        col0 = lax.iota(jnp.int32, 16) * _W

        def expand_and_fire(c, carry):
            x = idx_v[pl.ds(c * 16, 16)]
            pos0 = col0 + c * _CROWS
            for j in range(_W):
                plsc.store_scatter(exp_v, [pos0 + j], x + (_ROW_SHIFT + j))
            pltpu.async_copy(
                emb_hbm.at[exp_v.at[pl.ds(c * _CROWS, _CROWS)]],
                rows_v.at[pl.ds(c * _CROWS, _CROWS)],
                sem,
            )
            return carry

        lax.fori_loop(0, _CH, expand_and_fire, 0)
        pltpu.make_async_copy(emb_hbm.at[exp_v], rows_v, sem).wait()
        pltpu.sync_copy(rows_v, out_hbm.at[pl.ds(base * _W, _ROWS)])

    return gather_kernel


def kernel(inputs, embedding):
    out2d = _build_gather()(inputs.astype(jnp.int32), embedding)
    return out2d.reshape(_B, _W, _D)
